# baseline (speedup 1.0000x reference)
import functools

import jax
import jax.numpy as jnp
from jax import lax
from jax.experimental import pallas as pl
from jax.experimental.pallas import tpu as pltpu
from jax.experimental.pallas import tpu_sc as plsc

N = 10000
E = 320000
D_IN = 128
D_HID = 16
D_OUT = 128

NC = 2
NS = 16
NW = NC * NS
CH = 128
NB = 4
EPT = -(-E // NW)
NCH = -(-EPT // (CH * NB)) * NB
E_PAD = NW * NCH * CH
N_PAD = 10240
RPT = N_PAD // NS


def _seg_sum_kernel(with_deg: bool):
  out_type = [jax.ShapeDtypeStruct((NC, N_PAD, D_HID), jnp.float32)]
  scratch = [
      pltpu.VMEM((NCH, CH), jnp.int32),
      pltpu.VMEM((NCH, CH), jnp.int32),
      [pltpu.VMEM((CH, D_HID), jnp.float32) for _ in range(2)],
      pltpu.VMEM_SHARED((N_PAD, D_HID), jnp.float32),
      [pltpu.SemaphoreType.DMA for _ in range(2)],
      pltpu.SemaphoreType.DMA,
  ]
  if with_deg:
    out_type.append(jax.ShapeDtypeStruct((NC, N_PAD, D_HID), jnp.float32))
    scratch.append(pltpu.VMEM((CH, D_HID), jnp.float32))
    scratch.append(pltpu.VMEM_SHARED((N_PAD, D_HID), jnp.float32))

  mesh = plsc.VectorSubcoreMesh(core_axis_name="c", subcore_axis_name="s")

  def body(vals_hbm, srcs_hbm, dsts_hbm, zeros_hbm, ones_hbm, *rest):
    if with_deg:
      out, dout, src_v, dst_v, rows, acc, gsem, dsem, ones_v, dacc = rest
    else:
      out, src_v, dst_v, rows, acc, gsem, dsem = rest
    c = lax.axis_index("c")
    s = lax.axis_index("s")
    wid = c * NS + s
    r0 = s * RPT
    pltpu.sync_copy(zeros_hbm.at[pl.ds(r0, RPT)], acc.at[pl.ds(r0, RPT)])
    if with_deg:
      pltpu.sync_copy(zeros_hbm.at[pl.ds(r0, RPT)], dacc.at[pl.ds(r0, RPT)])
      pltpu.sync_copy(ones_hbm, ones_v)
    pltpu.sync_copy(srcs_hbm.at[wid], src_v)
    pltpu.sync_copy(dsts_hbm.at[wid], dst_v)
    plsc.subcore_barrier()

    def _gather(k, b):
      pltpu.async_copy(vals_hbm.at[src_v.at[k]], rows[b], gsem[b])

    def _consume(k, b):
      pltpu.make_async_copy(vals_hbm.at[src_v.at[0]], rows[b], gsem[b]).wait()
      pltpu.sync_copy(rows[b], acc.at[dst_v.at[k]], add=True)
      if with_deg:
        pltpu.sync_copy(ones_v, dacc.at[dst_v.at[k]], add=True)

    _gather(0, 0)

    @pl.loop(0, NCH, step=2)
    def _pair(j):
      _gather(j + 1, 1)
      _consume(j, 0)
      @pl.when(j + 2 < NCH)
      def _():
        _gather(j + 2, 0)
      _consume(j + 1, 1)


    plsc.subcore_barrier()
    pltpu.sync_copy(acc.at[pl.ds(r0, RPT)], out.at[c, pl.ds(r0, RPT)])
    if with_deg:
      pltpu.sync_copy(dacc.at[pl.ds(r0, RPT)], dout.at[c, pl.ds(r0, RPT)])

  return pl.kernel(
      body, out_type=out_type, mesh=mesh, scratch_types=scratch,
      compiler_params=pltpu.CompilerParams(use_tc_tiling_on_sc=False))


def _mm1_body(x_ref, w_ref, o_ref):
  o_ref[...] = jnp.dot(x_ref[...], w_ref[...],
                       preferred_element_type=jnp.float32)


def _mid_body(s1p_ref, dp_ref, y_ref, b1_ref, h1_ref, inv_ref):
  deg = dp_ref[0, :N, :] + dp_ref[1, :N, :]
  inv = 1.0 / (deg + 1.0)
  s1 = s1p_ref[0, :N, :] + s1p_ref[1, :N, :]
  h = (s1 + y_ref[...]) * inv + b1_ref[...]
  h1_ref[...] = jnp.maximum(h, 0.0)
  inv_ref[...] = inv


def _out_body(s2p_ref, h1_ref, inv_ref, w_ref, b2_ref, o_ref):
  t = (s2p_ref[0, :N, :] + s2p_ref[1, :N, :] + h1_ref[...]) * inv_ref[...]
  o_ref[...] = jnp.dot(t, w_ref[...],
                       preferred_element_type=jnp.float32) + b2_ref[...]


_seg_sum_deg = _seg_sum_kernel(with_deg=True)
_seg_sum = _seg_sum_kernel(with_deg=False)

_mm1 = pl.pallas_call(
    _mm1_body, out_shape=jax.ShapeDtypeStruct((N, D_HID), jnp.float32))

_mid = pl.pallas_call(
    _mid_body,
    out_shape=[jax.ShapeDtypeStruct((N, D_HID), jnp.float32),
               jax.ShapeDtypeStruct((N, D_HID), jnp.float32)])

_out = pl.pallas_call(
    _out_body, out_shape=jax.ShapeDtypeStruct((N, D_OUT), jnp.float32))


def kernel(x, edge_index, W1, b1, W2, b2):
  src = edge_index[0]
  dst = edge_index[1]
  ept = E // NW
  ppt = NCH * CH - ept
  pad_src = jnp.zeros((NW, ppt), jnp.int32)
  pad_dst = jnp.broadcast_to(N + jnp.arange(ppt, dtype=jnp.int32), (NW, ppt))
  srcs = jnp.concatenate(
      [src.reshape(NW, ept), pad_src], axis=1).reshape(NW, NCH, CH)
  dsts = jnp.concatenate(
      [dst.reshape(NW, ept), pad_dst], axis=1).reshape(NW, NCH, CH)
  zeros = jnp.zeros((N_PAD, D_HID), jnp.float32)
  ones = jnp.ones((CH, D_HID), jnp.float32)

  y = _mm1(x, W1)
  s1p, degp = _seg_sum_deg(y, srcs, dsts, zeros, ones)
  h1, inv = _mid(s1p, degp, y, b1.reshape(1, D_HID))
  (s2p,) = _seg_sum(h1, srcs, dsts, zeros, ones)
  out = _out(s2p, h1, inv, W2, b2.reshape(1, D_OUT))
  return out


# speedup vs baseline: 1.2108x; 1.2108x over previous
import functools

import jax
import jax.numpy as jnp
from jax import lax
from jax.experimental import pallas as pl
from jax.experimental.pallas import tpu as pltpu
from jax.experimental.pallas import tpu_sc as plsc

N = 10000
E = 320000
D_IN = 128
D_HID = 16
D_OUT = 128

NC = 2
NS = 16
NW = NC * NS
CH = 128
EPT = -(-E // NW)
NCH = -(-EPT // CH)
E_PAD = NW * NCH * CH
N_PAD = 10240
RPT = N_PAD // NS


def _seg_sum_kernel(with_deg: bool):
  out_type = [jax.ShapeDtypeStruct((NC, N_PAD, D_HID), jnp.float32)]
  scratch = [
      pltpu.VMEM((NCH, CH), jnp.int32),
      pltpu.VMEM((NCH, CH), jnp.int32),
      pltpu.VMEM((CH, D_HID), jnp.float32),
      pltpu.VMEM((CH, D_HID), jnp.float32),
      pltpu.VMEM_SHARED((N_PAD, D_HID), jnp.float32),
      pltpu.SemaphoreType.DMA,
      pltpu.SemaphoreType.DMA,
  ]
  if with_deg:
    out_type.append(jax.ShapeDtypeStruct((NC, N_PAD, D_HID), jnp.float32))
    scratch.append(pltpu.VMEM((CH, D_HID), jnp.float32))
    scratch.append(pltpu.VMEM_SHARED((N_PAD, D_HID), jnp.float32))

  mesh = plsc.VectorSubcoreMesh(core_axis_name="c", subcore_axis_name="s")

  def body(vals_hbm, srcs_hbm, dsts_hbm, zeros_hbm, ones_hbm, *rest):
    if with_deg:
      out, dout, src_v, dst_v, rows_a, rows_b, acc, sem_a, sem_b, ones_v, dacc = rest
    else:
      out, src_v, dst_v, rows_a, rows_b, acc, sem_a, sem_b = rest
    c = lax.axis_index("c")
    s = lax.axis_index("s")
    wid = c * NS + s
    r0 = s * RPT
    pltpu.sync_copy(zeros_hbm.at[pl.ds(r0, RPT)], acc.at[pl.ds(r0, RPT)])
    if with_deg:
      pltpu.sync_copy(zeros_hbm.at[pl.ds(r0, RPT)], dacc.at[pl.ds(r0, RPT)])
      pltpu.sync_copy(ones_hbm, ones_v)
    pltpu.sync_copy(srcs_hbm.at[wid], src_v)
    pltpu.sync_copy(dsts_hbm.at[wid], dst_v)
    plsc.subcore_barrier()

    def _gather(j, buf, sem):
      pltpu.async_copy(vals_hbm.at[src_v.at[j]], buf, sem)

    def _consume(j, buf, sem):
      pltpu.make_async_copy(vals_hbm.at[src_v.at[j]], buf, sem).wait()
      pltpu.sync_copy(buf, acc.at[dst_v.at[j]], add=True)
      if with_deg:
        pltpu.sync_copy(ones_v, dacc.at[dst_v.at[j]], add=True)

    _gather(0, rows_a, sem_a)

    @pl.loop(0, NCH, step=2)
    def _pair(j):
      @pl.when(j + 1 < NCH)
      def _():
        _gather(j + 1, rows_b, sem_b)
      _consume(j, rows_a, sem_a)

      @pl.when(j + 2 < NCH)
      def _():
        _gather(j + 2, rows_a, sem_a)

      @pl.when(j + 1 < NCH)
      def _():
        _consume(j + 1, rows_b, sem_b)

    plsc.subcore_barrier()
    pltpu.sync_copy(acc.at[pl.ds(r0, RPT)], out.at[c, pl.ds(r0, RPT)])
    if with_deg:
      pltpu.sync_copy(dacc.at[pl.ds(r0, RPT)], dout.at[c, pl.ds(r0, RPT)])

  return pl.kernel(
      body, out_type=out_type, mesh=mesh, scratch_types=scratch,
      compiler_params=pltpu.CompilerParams(use_tc_tiling_on_sc=False))


def _mm1_body(x_ref, w_ref, o_ref):
  o_ref[...] = jnp.dot(x_ref[...], w_ref[...],
                       preferred_element_type=jnp.float32)


def _mid_body(s1p_ref, dp_ref, y_ref, b1_ref, h1_ref, inv_ref):
  deg = dp_ref[0, :N, :] + dp_ref[1, :N, :]
  inv = 1.0 / (deg + 1.0)
  s1 = s1p_ref[0, :N, :] + s1p_ref[1, :N, :]
  h = (s1 + y_ref[...]) * inv + b1_ref[...]
  h1_ref[...] = jnp.maximum(h, 0.0)
  inv_ref[...] = inv


def _out_body(s2p_ref, h1_ref, inv_ref, w_ref, b2_ref, o_ref):
  t = (s2p_ref[0, :N, :] + s2p_ref[1, :N, :] + h1_ref[...]) * inv_ref[...]
  o_ref[...] = jnp.dot(t, w_ref[...],
                       preferred_element_type=jnp.float32) + b2_ref[...]


_seg_sum_deg = _seg_sum_kernel(with_deg=True)
_seg_sum = _seg_sum_kernel(with_deg=False)

_mm1 = pl.pallas_call(
    _mm1_body, out_shape=jax.ShapeDtypeStruct((N, D_HID), jnp.float32))

_mid = pl.pallas_call(
    _mid_body,
    out_shape=[jax.ShapeDtypeStruct((N, D_HID), jnp.float32),
               jax.ShapeDtypeStruct((N, D_HID), jnp.float32)])

_out = pl.pallas_call(
    _out_body, out_shape=jax.ShapeDtypeStruct((N, D_OUT), jnp.float32))


def kernel(x, edge_index, W1, b1, W2, b2):
  src = edge_index[0]
  dst = edge_index[1]
  pad = E_PAD - E
  srcs = jnp.concatenate(
      [src, jnp.zeros((pad,), jnp.int32)]).reshape(NW, NCH, CH)
  dsts = jnp.concatenate(
      [dst, jnp.full((pad,), N, jnp.int32)]).reshape(NW, NCH, CH)
  zeros = jnp.zeros((N_PAD, D_HID), jnp.float32)
  ones = jnp.ones((CH, D_HID), jnp.float32)

  y = _mm1(x, W1)
  s1p, degp = _seg_sum_deg(y, srcs, dsts, zeros, ones)
  h1, inv = _mid(s1p, degp, y, b1.reshape(1, D_HID))
  (s2p,) = _seg_sum(h1, srcs, dsts, zeros, ones)
  out = _out(s2p, h1, inv, W2, b2.reshape(1, D_OUT))
  return out


# speedup vs baseline: 1.3762x; 1.1366x over previous
import functools

import jax
import jax.numpy as jnp
from jax import lax
from jax.experimental import pallas as pl
from jax.experimental.pallas import tpu as pltpu
from jax.experimental.pallas import tpu_sc as plsc

N = 10000
E = 320000
D_IN = 128
D_HID = 16
D_OUT = 128

NC = 2
NS = 16
NW = NC * NS
CH = 128
EPT = -(-E // NW)
NCH = -(-EPT // CH)
E_PAD = NW * NCH * CH
N_PAD = 10240
RPT = N_PAD // NS


def _seg_sum_kernel(with_deg: bool):
  out_type = [jax.ShapeDtypeStruct((NC, N_PAD, D_HID), jnp.float32)]
  scratch = [
      pltpu.VMEM((NCH, CH), jnp.int32),
      pltpu.VMEM((NCH, CH), jnp.int32),
      pltpu.VMEM((CH, D_HID), jnp.float32),
      pltpu.VMEM((CH, D_HID), jnp.float32),
      pltpu.VMEM_SHARED((N_PAD, D_HID), jnp.float32),
      pltpu.SemaphoreType.DMA,
      pltpu.SemaphoreType.DMA,
  ]
  if with_deg:
    out_type.append(jax.ShapeDtypeStruct((NC, N_PAD, D_HID), jnp.float32))
    scratch.append(pltpu.VMEM((CH, D_HID), jnp.float32))
    scratch.append(pltpu.VMEM_SHARED((N_PAD, D_HID), jnp.float32))

  mesh = plsc.VectorSubcoreMesh(core_axis_name="c", subcore_axis_name="s")

  def body(vals_hbm, srcs_hbm, dsts_hbm, zeros_hbm, ones_hbm, *rest):
    if with_deg:
      out, dout, src_v, dst_v, rows_a, rows_b, acc, sem_a, sem_b, ones_v, dacc = rest
    else:
      out, src_v, dst_v, rows_a, rows_b, acc, sem_a, sem_b = rest
    c = lax.axis_index("c")
    s = lax.axis_index("s")
    wid = c * NS + s
    r0 = s * RPT
    pltpu.sync_copy(zeros_hbm.at[pl.ds(r0, RPT)], acc.at[pl.ds(r0, RPT)])
    if with_deg:
      pltpu.sync_copy(zeros_hbm.at[pl.ds(r0, RPT)], dacc.at[pl.ds(r0, RPT)])
      pltpu.sync_copy(ones_hbm, ones_v)
    pltpu.sync_copy(srcs_hbm.at[wid], src_v)
    pltpu.sync_copy(dsts_hbm.at[wid], dst_v)
    plsc.subcore_barrier()

    def _gather(j, buf, sem):
      pltpu.async_copy(vals_hbm.at[src_v.at[j]], buf, sem)

    def _consume(j, buf, sem):
      pltpu.make_async_copy(vals_hbm.at[src_v.at[j]], buf, sem).wait()
      pltpu.sync_copy(buf, acc.at[dst_v.at[j]], add=True)
      if with_deg:
        pltpu.sync_copy(ones_v, dacc.at[dst_v.at[j]], add=True)

    _gather(0, rows_a, sem_a)

    @pl.loop(0, NCH, step=2)
    def _pair(j):
      @pl.when(j + 1 < NCH)
      def _():
        _gather(j + 1, rows_b, sem_b)
      _consume(j, rows_a, sem_a)

      @pl.when(j + 2 < NCH)
      def _():
        _gather(j + 2, rows_a, sem_a)

      @pl.when(j + 1 < NCH)
      def _():
        _consume(j + 1, rows_b, sem_b)

    plsc.subcore_barrier()
    pltpu.sync_copy(acc.at[pl.ds(r0, RPT)], out.at[c, pl.ds(r0, RPT)])
    if with_deg:
      pltpu.sync_copy(dacc.at[pl.ds(r0, RPT)], dout.at[c, pl.ds(r0, RPT)])

  return pl.kernel(
      body, out_type=out_type, mesh=mesh, scratch_types=scratch,
      compiler_params=pltpu.CompilerParams(use_tc_tiling_on_sc=False))


def _mm1_body(x_ref, w_ref, o_ref):
  o_ref[...] = jnp.dot(x_ref[...], w_ref[...],
                       preferred_element_type=jnp.float32)


def _mid_body(s1p_ref, dp_ref, y_ref, b1_ref, h1_ref, inv_ref):
  deg = dp_ref[0, :N, :] + dp_ref[1, :N, :]
  inv = 1.0 / (deg + 1.0)
  s1 = s1p_ref[0, :N, :] + s1p_ref[1, :N, :]
  h = (s1 + y_ref[...]) * inv + b1_ref[...]
  h1_ref[...] = jnp.maximum(h, 0.0)
  inv_ref[...] = inv


def _out_body(s2p_ref, h1_ref, inv_ref, w_ref, b2_ref, o_ref):
  t = (s2p_ref[0, :N, :] + s2p_ref[1, :N, :] + h1_ref[...]) * inv_ref[...]
  o_ref[...] = jnp.dot(t, w_ref[...],
                       preferred_element_type=jnp.float32) + b2_ref[...]


_seg_sum_deg = _seg_sum_kernel(with_deg=True)
_seg_sum = _seg_sum_kernel(with_deg=False)

_mm1 = pl.pallas_call(
    _mm1_body, out_shape=jax.ShapeDtypeStruct((N, D_HID), jnp.float32))

_mid = pl.pallas_call(
    _mid_body,
    out_shape=[jax.ShapeDtypeStruct((N, D_HID), jnp.float32),
               jax.ShapeDtypeStruct((N, D_HID), jnp.float32)])

_out = pl.pallas_call(
    _out_body, out_shape=jax.ShapeDtypeStruct((N, D_OUT), jnp.float32))


def kernel(x, edge_index, W1, b1, W2, b2):
  src = edge_index[0]
  dst = edge_index[1]
  pad = E_PAD - E
  pad_ar = jnp.arange(pad, dtype=jnp.int32)
  srcs = jnp.concatenate(
      [src, pad_ar % N]).reshape(NW, NCH, CH)
  dsts = jnp.concatenate(
      [dst, N + pad_ar % (N_PAD - N)]).reshape(NW, NCH, CH)
  zeros = jnp.zeros((N_PAD, D_HID), jnp.float32)
  ones = jnp.ones((CH, D_HID), jnp.float32)

  y = _mm1(x, W1)
  s1p, degp = _seg_sum_deg(y, srcs, dsts, zeros, ones)
  h1, inv = _mid(s1p, degp, y, b1.reshape(1, D_HID))
  (s2p,) = _seg_sum(h1, srcs, dsts, zeros, ones)
  out = _out(s2p, h1, inv, W2, b2.reshape(1, D_OUT))
  return out


# speedup vs baseline: 1.4823x; 1.0771x over previous
import functools

import jax
import jax.numpy as jnp
from jax import lax
from jax.experimental import pallas as pl
from jax.experimental.pallas import tpu as pltpu
from jax.experimental.pallas import tpu_sc as plsc

N = 10000
E = 320000
D_IN = 128
D_HID = 16
D_OUT = 128

NC = 2
NS = 16
NW = NC * NS
CH = 128
NB = 4
EPT = -(-E // NW)
NCH = -(-EPT // (CH * NB)) * NB
E_PAD = NW * NCH * CH
N_PAD = 10240
RPT = N_PAD // NS


def _seg_sum_kernel(with_deg: bool):
  out_type = [jax.ShapeDtypeStruct((NC, N_PAD, D_HID), jnp.float32)]
  scratch = [
      pltpu.VMEM((NCH, CH), jnp.int32),
      pltpu.VMEM((NCH, CH), jnp.int32),
      [pltpu.VMEM((CH, D_HID), jnp.float32) for _ in range(NB)],
      pltpu.VMEM_SHARED((N_PAD, D_HID), jnp.float32),
      [pltpu.SemaphoreType.DMA for _ in range(NB)],
      [pltpu.SemaphoreType.DMA for _ in range(NB)],
  ]
  if with_deg:
    out_type.append(jax.ShapeDtypeStruct((NC, N_PAD, D_HID), jnp.float32))
    scratch.append(pltpu.VMEM((CH, D_HID), jnp.float32))
    scratch.append(pltpu.VMEM_SHARED((N_PAD, D_HID), jnp.float32))

  mesh = plsc.VectorSubcoreMesh(core_axis_name="c", subcore_axis_name="s")

  def body(vals_hbm, srcs_hbm, dsts_hbm, zeros_hbm, ones_hbm, *rest):
    if with_deg:
      out, dout, src_v, dst_v, rows, acc, gsem, ssem, ones_v, dacc = rest
    else:
      out, src_v, dst_v, rows, acc, gsem, ssem = rest
    c = lax.axis_index("c")
    s = lax.axis_index("s")
    wid = c * NS + s
    r0 = s * RPT
    pltpu.sync_copy(zeros_hbm.at[pl.ds(r0, RPT)], acc.at[pl.ds(r0, RPT)])
    if with_deg:
      pltpu.sync_copy(zeros_hbm.at[pl.ds(r0, RPT)], dacc.at[pl.ds(r0, RPT)])
      pltpu.sync_copy(ones_hbm, ones_v)
    pltpu.sync_copy(srcs_hbm.at[wid], src_v)
    pltpu.sync_copy(dsts_hbm.at[wid], dst_v)
    plsc.subcore_barrier()

    def _gather(k, b):
      pltpu.async_copy(vals_hbm.at[src_v.at[k]], rows[b], gsem[b])

    def _wait_gather(b):
      pltpu.make_async_copy(vals_hbm.at[src_v.at[0]], rows[b], gsem[b]).wait()

    def _scatter(k, b):
      pltpu.async_copy(rows[b], acc.at[dst_v.at[k]], ssem[b], add=True)
      if with_deg:
        pltpu.async_copy(ones_v, dacc.at[dst_v.at[k]], ssem[b], add=True)

    def _wait_scatter(b):
      pltpu.make_async_copy(rows[b], acc.at[dst_v.at[0]], ssem[b]).wait()
      if with_deg:
        pltpu.make_async_copy(ones_v, dacc.at[dst_v.at[0]], ssem[b]).wait()

    _gather(0, 0)
    _gather(1, 1)

    @pl.loop(0, NCH, step=NB)
    def _ring(j):
      for u in range(NB):
        k = j + u
        b = u
        bn = (u + 2) % NB
        _wait_gather(b)
        _scatter(k, b)

        @pl.when(k + 2 < NCH)
        def _():
          @pl.when(k >= 2)
          def _():
            _wait_scatter(bn)
          _gather(k + 2, bn)

    for b in range(NB):
      _wait_scatter(b)

    plsc.subcore_barrier()
    pltpu.sync_copy(acc.at[pl.ds(r0, RPT)], out.at[c, pl.ds(r0, RPT)])
    if with_deg:
      pltpu.sync_copy(dacc.at[pl.ds(r0, RPT)], dout.at[c, pl.ds(r0, RPT)])

  return pl.kernel(
      body, out_type=out_type, mesh=mesh, scratch_types=scratch,
      compiler_params=pltpu.CompilerParams(use_tc_tiling_on_sc=False))


def _mm1_body(x_ref, w_ref, o_ref):
  o_ref[...] = jnp.dot(x_ref[...], w_ref[...],
                       preferred_element_type=jnp.float32)


def _mid_body(s1p_ref, dp_ref, y_ref, b1_ref, h1_ref, inv_ref):
  deg = dp_ref[0, :N, :] + dp_ref[1, :N, :]
  inv = 1.0 / (deg + 1.0)
  s1 = s1p_ref[0, :N, :] + s1p_ref[1, :N, :]
  h = (s1 + y_ref[...]) * inv + b1_ref[...]
  h1_ref[...] = jnp.maximum(h, 0.0)
  inv_ref[...] = inv


def _out_body(s2p_ref, h1_ref, inv_ref, w_ref, b2_ref, o_ref):
  t = (s2p_ref[0, :N, :] + s2p_ref[1, :N, :] + h1_ref[...]) * inv_ref[...]
  o_ref[...] = jnp.dot(t, w_ref[...],
                       preferred_element_type=jnp.float32) + b2_ref[...]


_seg_sum_deg = _seg_sum_kernel(with_deg=True)
_seg_sum = _seg_sum_kernel(with_deg=False)

_mm1 = pl.pallas_call(
    _mm1_body, out_shape=jax.ShapeDtypeStruct((N, D_HID), jnp.float32))

_mid = pl.pallas_call(
    _mid_body,
    out_shape=[jax.ShapeDtypeStruct((N, D_HID), jnp.float32),
               jax.ShapeDtypeStruct((N, D_HID), jnp.float32)])

_out = pl.pallas_call(
    _out_body, out_shape=jax.ShapeDtypeStruct((N, D_OUT), jnp.float32))


def kernel(x, edge_index, W1, b1, W2, b2):
  src = edge_index[0]
  dst = edge_index[1]
  pad = E_PAD - E
  pad_ar = jnp.arange(pad, dtype=jnp.int32)
  srcs = jnp.concatenate(
      [src, pad_ar % N]).reshape(NW, NCH, CH)
  dsts = jnp.concatenate(
      [dst, N + pad_ar % (N_PAD - N)]).reshape(NW, NCH, CH)
  zeros = jnp.zeros((N_PAD, D_HID), jnp.float32)
  ones = jnp.ones((CH, D_HID), jnp.float32)

  y = _mm1(x, W1)
  s1p, degp = _seg_sum_deg(y, srcs, dsts, zeros, ones)
  h1, inv = _mid(s1p, degp, y, b1.reshape(1, D_HID))
  (s2p,) = _seg_sum(h1, srcs, dsts, zeros, ones)
  out = _out(s2p, h1, inv, W2, b2.reshape(1, D_OUT))
  return out


# speedup vs baseline: 1.6916x; 1.1411x over previous
import functools

import jax
import jax.numpy as jnp
from jax import lax
from jax.experimental import pallas as pl
from jax.experimental.pallas import tpu as pltpu
from jax.experimental.pallas import tpu_sc as plsc

N = 10000
E = 320000
D_IN = 128
D_HID = 16
D_OUT = 128

NC = 2
NS = 16
NW = NC * NS
CH = 256
NB = 4
EPT = -(-E // NW)
NCH = -(-EPT // (CH * NB)) * NB
E_PAD = NW * NCH * CH
N_PAD = 10240
RPT = N_PAD // NS


def _seg_sum_kernel(with_deg: bool):
  out_type = [jax.ShapeDtypeStruct((NC, N_PAD, D_HID), jnp.float32)]
  scratch = [
      pltpu.VMEM((NCH, CH), jnp.int32),
      pltpu.VMEM((NCH, CH), jnp.int32),
      [pltpu.VMEM((CH, D_HID), jnp.float32) for _ in range(NB)],
      pltpu.VMEM_SHARED((N_PAD, D_HID), jnp.float32),
      [pltpu.SemaphoreType.DMA for _ in range(NB)],
      [pltpu.SemaphoreType.DMA for _ in range(NB)],
  ]
  if with_deg:
    out_type.append(jax.ShapeDtypeStruct((NC, N_PAD, D_HID), jnp.float32))
    scratch.append(pltpu.VMEM((CH, D_HID), jnp.float32))
    scratch.append(pltpu.VMEM_SHARED((N_PAD, D_HID), jnp.float32))

  mesh = plsc.VectorSubcoreMesh(core_axis_name="c", subcore_axis_name="s")

  def body(vals_hbm, srcs_hbm, dsts_hbm, zeros_hbm, ones_hbm, *rest):
    if with_deg:
      out, dout, src_v, dst_v, rows, acc, gsem, ssem, ones_v, dacc = rest
    else:
      out, src_v, dst_v, rows, acc, gsem, ssem = rest
    c = lax.axis_index("c")
    s = lax.axis_index("s")
    wid = c * NS + s
    r0 = s * RPT
    pltpu.sync_copy(zeros_hbm.at[pl.ds(r0, RPT)], acc.at[pl.ds(r0, RPT)])
    if with_deg:
      pltpu.sync_copy(zeros_hbm.at[pl.ds(r0, RPT)], dacc.at[pl.ds(r0, RPT)])
      pltpu.sync_copy(ones_hbm, ones_v)
    pltpu.sync_copy(srcs_hbm.at[wid], src_v)
    pltpu.sync_copy(dsts_hbm.at[wid], dst_v)
    plsc.subcore_barrier()

    def _gather(k, b):
      pltpu.async_copy(vals_hbm.at[src_v.at[k]], rows[b], gsem[b])

    def _wait_gather(b):
      pltpu.make_async_copy(vals_hbm.at[src_v.at[0]], rows[b], gsem[b]).wait()

    def _scatter(k, b):
      pltpu.async_copy(rows[b], acc.at[dst_v.at[k]], ssem[b], add=True)
      if with_deg:
        pltpu.async_copy(ones_v, dacc.at[dst_v.at[k]], ssem[b], add=True)

    def _wait_scatter(b):
      pltpu.make_async_copy(rows[b], acc.at[dst_v.at[0]], ssem[b]).wait()
      if with_deg:
        pltpu.make_async_copy(ones_v, dacc.at[dst_v.at[0]], ssem[b]).wait()

    _gather(0, 0)
    _gather(1, 1)

    @pl.loop(0, NCH, step=NB)
    def _ring(j):
      for u in range(NB):
        k = j + u
        b = u
        bn = (u + 2) % NB
        _wait_gather(b)
        _scatter(k, b)

        @pl.when(k + 2 < NCH)
        def _():
          @pl.when(k >= 2)
          def _():
            _wait_scatter(bn)
          _gather(k + 2, bn)

    for b in range(NB):
      _wait_scatter(b)

    plsc.subcore_barrier()
    pltpu.sync_copy(acc.at[pl.ds(r0, RPT)], out.at[c, pl.ds(r0, RPT)])
    if with_deg:
      pltpu.sync_copy(dacc.at[pl.ds(r0, RPT)], dout.at[c, pl.ds(r0, RPT)])

  return pl.kernel(
      body, out_type=out_type, mesh=mesh, scratch_types=scratch,
      compiler_params=pltpu.CompilerParams(use_tc_tiling_on_sc=False))


def _mm1_body(x_ref, w_ref, o_ref):
  o_ref[...] = jnp.dot(x_ref[...], w_ref[...],
                       preferred_element_type=jnp.float32)


def _mid_body(s1p_ref, dp_ref, y_ref, b1_ref, h1_ref, inv_ref):
  deg = dp_ref[0, :N, :] + dp_ref[1, :N, :]
  inv = 1.0 / (deg + 1.0)
  s1 = s1p_ref[0, :N, :] + s1p_ref[1, :N, :]
  h = (s1 + y_ref[...]) * inv + b1_ref[...]
  h1_ref[...] = jnp.maximum(h, 0.0)
  inv_ref[...] = inv


def _out_body(s2p_ref, h1_ref, inv_ref, w_ref, b2_ref, o_ref):
  t = (s2p_ref[0, :N, :] + s2p_ref[1, :N, :] + h1_ref[...]) * inv_ref[...]
  o_ref[...] = jnp.dot(t, w_ref[...],
                       preferred_element_type=jnp.float32) + b2_ref[...]


_seg_sum_deg = _seg_sum_kernel(with_deg=True)
_seg_sum = _seg_sum_kernel(with_deg=False)

_mm1 = pl.pallas_call(
    _mm1_body, out_shape=jax.ShapeDtypeStruct((N, D_HID), jnp.float32))

_mid = pl.pallas_call(
    _mid_body,
    out_shape=[jax.ShapeDtypeStruct((N, D_HID), jnp.float32),
               jax.ShapeDtypeStruct((N, D_HID), jnp.float32)])

_out = pl.pallas_call(
    _out_body, out_shape=jax.ShapeDtypeStruct((N, D_OUT), jnp.float32))


def kernel(x, edge_index, W1, b1, W2, b2):
  src = edge_index[0]
  dst = edge_index[1]
  pad = E_PAD - E
  pad_ar = jnp.arange(pad, dtype=jnp.int32)
  srcs = jnp.concatenate(
      [src, pad_ar % N]).reshape(NW, NCH, CH)
  dsts = jnp.concatenate(
      [dst, N + pad_ar % (N_PAD - N)]).reshape(NW, NCH, CH)
  zeros = jnp.zeros((N_PAD, D_HID), jnp.float32)
  ones = jnp.ones((CH, D_HID), jnp.float32)

  y = _mm1(x, W1)
  s1p, degp = _seg_sum_deg(y, srcs, dsts, zeros, ones)
  h1, inv = _mid(s1p, degp, y, b1.reshape(1, D_HID))
  (s2p,) = _seg_sum(h1, srcs, dsts, zeros, ones)
  out = _out(s2p, h1, inv, W2, b2.reshape(1, D_OUT))
  return out


# speedup vs baseline: 1.7846x; 1.0550x over previous
import functools

import jax
import jax.numpy as jnp
from jax import lax
from jax.experimental import pallas as pl
from jax.experimental.pallas import tpu as pltpu
from jax.experimental.pallas import tpu_sc as plsc

N = 10000
E = 320000
D_IN = 128
D_HID = 16
D_OUT = 128

NC = 2
NS = 16
NW = NC * NS
CH = 512
NB = 4
EPT = -(-E // NW)
NCH = -(-EPT // (CH * NB)) * NB
E_PAD = NW * NCH * CH
N_PAD = 10240
RPT = N_PAD // NS


def _seg_sum_kernel(with_deg: bool):
  out_type = [jax.ShapeDtypeStruct((NC, N_PAD, D_HID), jnp.float32)]
  scratch = [
      pltpu.VMEM((NCH, CH), jnp.int32),
      pltpu.VMEM((NCH, CH), jnp.int32),
      [pltpu.VMEM((CH, D_HID), jnp.float32) for _ in range(NB)],
      pltpu.VMEM_SHARED((N_PAD, D_HID), jnp.float32),
      [pltpu.SemaphoreType.DMA for _ in range(NB)],
      [pltpu.SemaphoreType.DMA for _ in range(NB)],
  ]
  if with_deg:
    out_type.append(jax.ShapeDtypeStruct((NC, N_PAD, D_HID), jnp.float32))
    scratch.append(pltpu.VMEM((CH, D_HID), jnp.float32))
    scratch.append(pltpu.VMEM_SHARED((N_PAD, D_HID), jnp.float32))

  mesh = plsc.VectorSubcoreMesh(core_axis_name="c", subcore_axis_name="s")

  def body(vals_hbm, srcs_hbm, dsts_hbm, zeros_hbm, ones_hbm, *rest):
    if with_deg:
      out, dout, src_v, dst_v, rows, acc, gsem, ssem, ones_v, dacc = rest
    else:
      out, src_v, dst_v, rows, acc, gsem, ssem = rest
    c = lax.axis_index("c")
    s = lax.axis_index("s")
    wid = c * NS + s
    r0 = s * RPT
    pltpu.sync_copy(zeros_hbm.at[pl.ds(r0, RPT)], acc.at[pl.ds(r0, RPT)])
    if with_deg:
      pltpu.sync_copy(zeros_hbm.at[pl.ds(r0, RPT)], dacc.at[pl.ds(r0, RPT)])
      pltpu.sync_copy(ones_hbm, ones_v)
    pltpu.sync_copy(srcs_hbm.at[wid], src_v)
    pltpu.sync_copy(dsts_hbm.at[wid], dst_v)
    plsc.subcore_barrier()

    def _gather(k, b):
      pltpu.async_copy(vals_hbm.at[src_v.at[k]], rows[b], gsem[b])

    def _wait_gather(b):
      pltpu.make_async_copy(vals_hbm.at[src_v.at[0]], rows[b], gsem[b]).wait()

    def _scatter(k, b):
      pltpu.async_copy(rows[b], acc.at[dst_v.at[k]], ssem[b], add=True)
      if with_deg:
        pltpu.async_copy(ones_v, dacc.at[dst_v.at[k]], ssem[b], add=True)

    def _wait_scatter(b):
      pltpu.make_async_copy(rows[b], acc.at[dst_v.at[0]], ssem[b]).wait()
      if with_deg:
        pltpu.make_async_copy(ones_v, dacc.at[dst_v.at[0]], ssem[b]).wait()

    _gather(0, 0)
    _gather(1, 1)

    @pl.loop(0, NCH, step=NB)
    def _ring(j):
      for u in range(NB):
        k = j + u
        b = u
        bn = (u + 2) % NB
        _wait_gather(b)
        _scatter(k, b)

        @pl.when(k + 2 < NCH)
        def _():
          @pl.when(k >= 2)
          def _():
            _wait_scatter(bn)
          _gather(k + 2, bn)

    for b in range(NB):
      _wait_scatter(b)

    plsc.subcore_barrier()
    pltpu.sync_copy(acc.at[pl.ds(r0, RPT)], out.at[c, pl.ds(r0, RPT)])
    if with_deg:
      pltpu.sync_copy(dacc.at[pl.ds(r0, RPT)], dout.at[c, pl.ds(r0, RPT)])

  return pl.kernel(
      body, out_type=out_type, mesh=mesh, scratch_types=scratch,
      compiler_params=pltpu.CompilerParams(use_tc_tiling_on_sc=False))


def _mm1_body(x_ref, w_ref, o_ref):
  o_ref[...] = jnp.dot(x_ref[...], w_ref[...],
                       preferred_element_type=jnp.float32)


def _mid_body(s1p_ref, dp_ref, y_ref, b1_ref, h1_ref, inv_ref):
  deg = dp_ref[0, :N, :] + dp_ref[1, :N, :]
  inv = 1.0 / (deg + 1.0)
  s1 = s1p_ref[0, :N, :] + s1p_ref[1, :N, :]
  h = (s1 + y_ref[...]) * inv + b1_ref[...]
  h1_ref[...] = jnp.maximum(h, 0.0)
  inv_ref[...] = inv


def _out_body(s2p_ref, h1_ref, inv_ref, w_ref, b2_ref, o_ref):
  t = (s2p_ref[0, :N, :] + s2p_ref[1, :N, :] + h1_ref[...]) * inv_ref[...]
  o_ref[...] = jnp.dot(t, w_ref[...],
                       preferred_element_type=jnp.float32) + b2_ref[...]


_seg_sum_deg = _seg_sum_kernel(with_deg=True)
_seg_sum = _seg_sum_kernel(with_deg=False)

_mm1 = pl.pallas_call(
    _mm1_body, out_shape=jax.ShapeDtypeStruct((N, D_HID), jnp.float32))

_mid = pl.pallas_call(
    _mid_body,
    out_shape=[jax.ShapeDtypeStruct((N, D_HID), jnp.float32),
               jax.ShapeDtypeStruct((N, D_HID), jnp.float32)])

_out = pl.pallas_call(
    _out_body, out_shape=jax.ShapeDtypeStruct((N, D_OUT), jnp.float32))


def kernel(x, edge_index, W1, b1, W2, b2):
  src = edge_index[0]
  dst = edge_index[1]
  pad = E_PAD - E
  pad_ar = jnp.arange(pad, dtype=jnp.int32)
  srcs = jnp.concatenate(
      [src, pad_ar % N]).reshape(NW, NCH, CH)
  dsts = jnp.concatenate(
      [dst, N + pad_ar % (N_PAD - N)]).reshape(NW, NCH, CH)
  zeros = jnp.zeros((N_PAD, D_HID), jnp.float32)
  ones = jnp.ones((CH, D_HID), jnp.float32)

  y = _mm1(x, W1)
  s1p, degp = _seg_sum_deg(y, srcs, dsts, zeros, ones)
  h1, inv = _mid(s1p, degp, y, b1.reshape(1, D_HID))
  (s2p,) = _seg_sum(h1, srcs, dsts, zeros, ones)
  out = _out(s2p, h1, inv, W2, b2.reshape(1, D_OUT))
  return out
